# baseline (speedup 1.0000x reference)
import jax
import jax.numpy as jnp
from jax import lax
from jax.experimental import pallas as pl
from jax.experimental.pallas import tpu as pltpu

_T = 4096
_L = 64
_D = 256
_H = _T // 2
_CH = 16


def _compose(src, idx):
    return jnp.take_along_axis(src, idx, axis=1)


def _crf_body(feats_ref, w_ref, b_row_ref, start_ref,
              t_ref, tt_ref, end_col_ref,
              score_ref, pev_ref, pod_ref,
              em_ref, srows_ref, bpsa_ref, bpsb_ref, h0_ref, h1_ref):
    f32 = jnp.float32
    i32 = jnp.int32

    em_ref[...] = (
        jnp.dot(feats_ref[...], w_ref[...], preferred_element_type=f32)
        + b_row_ref[...]
    )

    liota_m = lax.broadcasted_iota(i32, (_L, _L), 1).astype(f32)
    siota_m = lax.broadcasted_iota(i32, (_L, _L), 0).astype(f32)
    siota_c = lax.broadcasted_iota(i32, (_L, 1), 0).astype(f32)

    tmat = t_ref[...]
    ttmat = tt_ref[...]

    s0 = start_ref[...] + em_ref[0:1, :]

    def fwd_pair(it, s_row):
        srows_ref[pl.ds(it, 1), :] = s_row
        swt = s_row + tmat
        mxa = jnp.max(swt, axis=1, keepdims=True)
        em_colb = jnp.broadcast_to(
            em_ref[pl.ds(2 * it + 1, 1), :].reshape(_L, 1), (_L, _L))
        s_colb = em_colb + mxa
        swt2 = s_colb + ttmat
        mxb = jnp.max(swt2, axis=0, keepdims=True)
        return em_ref[pl.ds(2 * it + 2, 1), :] + mxb

    s_row = lax.fori_loop(0, _H - 1, fwd_pair, s0)
    srows_ref[_H - 1:_H, :] = s_row

    swt = s_row + tmat
    mxa = jnp.max(swt, axis=1, keepdims=True)
    em_col = em_ref[_T - 1:_T, :].reshape(_L, 1)
    final = (em_col + mxa) + end_col_ref[...]
    vs = jnp.max(final)
    score_ref[...] = jnp.full((1, 1), vs, f32)
    last = jnp.min(jnp.where(final == vs, siota_c, float(_L)),
                   axis=0, keepdims=True).astype(i32)

    def bp_chunk(c, _):
        for u in range(_CH):
            idx = c * _CH + u
            s_row_m = srows_ref[pl.ds(idx, 1), :]
            swt = s_row_m + tmat
            mxa_m = jnp.max(swt, axis=1, keepdims=True)
            bpa = jnp.min(jnp.where(swt == mxa_m, liota_m, float(_L)),
                          axis=1, keepdims=True)
            bpsa_ref[pl.ds(idx, 1), :] = bpa.reshape(1, _L).astype(i32)
            em_colb_m = jnp.broadcast_to(
                em_ref[pl.ds(2 * idx + 1, 1), :].reshape(_L, 1), (_L, _L))
            swt2 = (em_colb_m + mxa_m) + ttmat
            mxb_m = jnp.max(swt2, axis=0, keepdims=True)
            bpb = jnp.min(jnp.where(swt2 == mxb_m, siota_m, float(_L)),
                          axis=0, keepdims=True)
            bpsb_ref[pl.ds(idx, 1), :] = bpb.astype(i32)
        return 0

    lax.fori_loop(0, _H // _CH, bp_chunk, 0)
    bpsb_ref[_H - 1:_H, :] = lax.broadcasted_iota(i32, (1, _L), 1)

    h0_ref[...] = _compose(bpsa_ref[...], bpsb_ref[...])

    src, dst = h0_ref, h1_ref
    off = 1
    while off < _H:
        n = _H - off
        dst[0:n, :] = _compose(src[0:n, :], src[off:_H, :])
        dst[n:_H, :] = src[n:_H, :]
        src, dst = dst, src
        off *= 2

    hodd = _compose(
        bpsb_ref[...],
        jnp.concatenate([src[1:_H, :],
                         lax.broadcasted_iota(i32, (1, _L), 1)], axis=0))

    idx_ev = jnp.broadcast_to(last, (_H, 1))
    pev_ref[...] = jnp.take_along_axis(src[...], idx_ev, axis=1)
    pod_ref[...] = jnp.take_along_axis(hodd, idx_ev, axis=1)


def kernel(feats, batch_sizes, W, b, start_transition, transitions,
           end_transition):
    del batch_sizes
    score, pev, pod = pl.pallas_call(
        _crf_body,
        out_shape=[
            jax.ShapeDtypeStruct((1, 1), jnp.float32),
            jax.ShapeDtypeStruct((_H, 1), jnp.int32),
            jax.ShapeDtypeStruct((_H, 1), jnp.int32),
        ],
        scratch_shapes=[
            pltpu.VMEM((_T, _L), jnp.float32),
            pltpu.VMEM((_H, _L), jnp.float32),
            pltpu.VMEM((_H, _L), jnp.int32),
            pltpu.VMEM((_H, _L), jnp.int32),
            pltpu.VMEM((_H, _L), jnp.int32),
            pltpu.VMEM((_H, _L), jnp.int32),
        ],
    )(
        feats,
        W,
        b.reshape(1, _L),
        start_transition.reshape(1, _L),
        transitions,
        transitions.T,
        end_transition.reshape(_L, 1),
    )
    path = jnp.stack([pev[:, 0], pod[:, 0]], axis=1).reshape(_T)
    return score[0, 0], path


# speedup vs baseline: 1.8012x; 1.8012x over previous
import jax
import jax.numpy as jnp
from jax import lax
from jax.experimental import pallas as pl
from jax.experimental.pallas import tpu as pltpu

_T = 4096
_L = 64
_D = 256
_H = _T // 2
_MT = 128
_NT = _H // _MT


def _compose(src, idx):
    return jnp.take_along_axis(src, idx, axis=1)


def _argmax_pass(state_t_ref, tb_ref, out_idx_ref, out_mx_ref):
    def tile_body(t, _):
        base = pl.multiple_of(t * _MT, _MT)
        acc = jnp.full((_L, _MT), -jnp.inf, jnp.float32)
        idx = jnp.zeros((_L, _MT), jnp.int32)
        for j in range(_L):
            c = tb_ref[64 * j:64 * j + 64, :] + \
                state_t_ref[j:j + 1, pl.ds(base, _MT)]
            gt = c > acc
            acc = jnp.where(gt, c, acc)
            idx = jnp.where(gt, j, idx)
        out_idx_ref[:, pl.ds(base, _MT)] = idx
        if out_mx_ref is not None:
            out_mx_ref[:, pl.ds(base, _MT)] = acc
        return 0

    lax.fori_loop(0, _NT, tile_body, 0)


def _crf_body(feats_ref, w_ref, b_row_ref, start_ref,
              t_ref, tt_ref, end_col_ref,
              score_ref, pev_ref, pod_ref,
              em_ref, srows_ref, emodd_ref, st_ref, tb_ref, mxat_ref,
              bpat_ref, bpbt_ref, bpsa_ref, bpsb_ref, h0_ref, h1_ref):
    f32 = jnp.float32
    i32 = jnp.int32

    em_ref[...] = (
        jnp.dot(feats_ref[...], w_ref[...], preferred_element_type=f32)
        + b_row_ref[...]
    )

    siota_c = lax.broadcasted_iota(i32, (_L, 1), 0).astype(f32)

    tmat = t_ref[...]
    ttmat = tt_ref[...]

    s0 = start_ref[...] + em_ref[0:1, :]

    def fwd_pair(it, s_row):
        srows_ref[pl.ds(it, 1), :] = s_row
        swt = s_row + tmat
        mxa = jnp.max(swt, axis=1, keepdims=True)
        em_row = em_ref[pl.ds(2 * it + 1, 1), :]
        emodd_ref[pl.ds(it, 1), :] = em_row
        em_colb = jnp.broadcast_to(em_row.reshape(_L, 1), (_L, _L))
        s_colb = em_colb + mxa
        swt2 = s_colb + ttmat
        mxb = jnp.max(swt2, axis=0, keepdims=True)
        return em_ref[pl.ds(2 * it + 2, 1), :] + mxb

    s_row = lax.fori_loop(0, _H - 1, fwd_pair, s0)
    srows_ref[_H - 1:_H, :] = s_row
    emodd_ref[_H - 1:_H, :] = em_ref[_T - 1:_T, :]

    swt = s_row + tmat
    mxa = jnp.max(swt, axis=1, keepdims=True)
    em_col = em_ref[_T - 1:_T, :].reshape(_L, 1)
    final = (em_col + mxa) + end_col_ref[...]
    vs = jnp.max(final)
    score_ref[...] = jnp.full((1, 1), vs, f32)
    last = jnp.min(jnp.where(final == vs, siota_c, float(_L)),
                   axis=0, keepdims=True).astype(i32)

    for j in range(_L):
        tb_ref[64 * j:64 * j + 64, :] = jnp.broadcast_to(
            t_ref[:, j:j + 1], (_L, _MT))

    st_ref[...] = srows_ref[...].T
    _argmax_pass(st_ref, tb_ref, bpat_ref, mxat_ref)

    st_ref[...] = emodd_ref[...].T + mxat_ref[...]
    _argmax_pass(st_ref, tb_ref, bpbt_ref, None)

    bpsa_ref[...] = bpat_ref[...].T
    bpsb_ref[...] = bpbt_ref[...].T
    bpsb_ref[_H - 1:_H, :] = lax.broadcasted_iota(i32, (1, _L), 1)

    h0_ref[...] = _compose(bpsa_ref[...], bpsb_ref[...])

    src, dst = h0_ref, h1_ref
    off = 1
    while off < _H:
        n = _H - off
        dst[0:n, :] = _compose(src[0:n, :], src[off:_H, :])
        dst[n:_H, :] = src[n:_H, :]
        src, dst = dst, src
        off *= 2

    hodd = _compose(
        bpsb_ref[...],
        jnp.concatenate([src[1:_H, :],
                         lax.broadcasted_iota(i32, (1, _L), 1)], axis=0))

    idx_ev = jnp.broadcast_to(last, (_H, 1))
    pev_ref[...] = jnp.take_along_axis(src[...], idx_ev, axis=1)
    pod_ref[...] = jnp.take_along_axis(hodd, idx_ev, axis=1)


def kernel(feats, batch_sizes, W, b, start_transition, transitions,
           end_transition):
    del batch_sizes
    score, pev, pod = pl.pallas_call(
        _crf_body,
        out_shape=[
            jax.ShapeDtypeStruct((1, 1), jnp.float32),
            jax.ShapeDtypeStruct((_H, 1), jnp.int32),
            jax.ShapeDtypeStruct((_H, 1), jnp.int32),
        ],
        scratch_shapes=[
            pltpu.VMEM((_T, _L), jnp.float32),
            pltpu.VMEM((_H, _L), jnp.float32),
            pltpu.VMEM((_H, _L), jnp.float32),
            pltpu.VMEM((_L, _H), jnp.float32),
            pltpu.VMEM((_L * _L, _MT), jnp.float32),
            pltpu.VMEM((_L, _H), jnp.float32),
            pltpu.VMEM((_L, _H), jnp.int32),
            pltpu.VMEM((_L, _H), jnp.int32),
            pltpu.VMEM((_H, _L), jnp.int32),
            pltpu.VMEM((_H, _L), jnp.int32),
            pltpu.VMEM((_H, _L), jnp.int32),
            pltpu.VMEM((_H, _L), jnp.int32),
        ],
    )(
        feats,
        W,
        b.reshape(1, _L),
        start_transition.reshape(1, _L),
        transitions,
        transitions.T,
        end_transition.reshape(_L, 1),
    )
    path = jnp.stack([pev[:, 0], pod[:, 0]], axis=1).reshape(_T)
    return score[0, 0], path
